# baseline (speedup 1.0000x reference)
import jax
import jax.numpy as jnp
from jax import lax
from jax.experimental import pallas as pl
from jax.experimental.pallas import tpu as pltpu
from jax.experimental.pallas import tpu_sc as plsc

_ROWS = 512
_COLS = 512
_KS = 5
_WI = _KS // 2

_NC = 2
_NS = 16
_NW = _NC * _NS
_RPW = _ROWS // _NW
_L = 16
_NCHUNK = _COLS // _L
_XROWS = _RPW + 2 * _WI
_CPAD = 8
_XCOLS = _COLS + 2 * _CPAD


def _sc_body(x_hbm, w_hbm, o_hbm, xbuf, wbuf, obuf):
    cid = lax.axis_index("c")
    sid = lax.axis_index("s")
    wid = sid * _NC + cid
    row0 = wid * _RPW

    zero = jnp.zeros((_L,), jnp.float32)
    for t in range(_XROWS):
        xbuf[t, pl.ds(0, _L)] = zero
        xbuf[t, pl.ds(_XCOLS - _L, _L)] = zero

    pltpu.sync_copy(
        x_hbm.at[pl.ds(row0, _XROWS), :],
        xbuf.at[pl.ds(0, _XROWS), pl.ds(_CPAD, _COLS)])
    pltpu.sync_copy(w_hbm, wbuf)

    wv = [wbuf[i, :] for i in range(_KS * _KS)]

    def row_body(r, carry):
        for c in range(_NCHUNK):
            acc = None
            for a in range(_KS):
                t = r + 2 * _WI - a
                for b in range(_KS):
                    off = c * _L + _CPAD + _WI - b
                    chunk = xbuf[t, pl.ds(off, _L)]
                    term = wv[a * _KS + b] * chunk
                    acc = term if acc is None else acc + term
            obuf[r, pl.ds(c * _L, _L)] = acc
        return carry

    lax.fori_loop(0, _RPW, row_body, 0)

    pltpu.sync_copy(obuf, o_hbm.at[pl.ds(row0, _RPW), :])


@jax.jit
def _splat_sc(x, wvec):
    xp = jnp.zeros((_ROWS + 2 * _WI, _COLS), jnp.float32)
    xp = lax.dynamic_update_slice(xp, x, (_WI, 0))
    mesh = plsc.VectorSubcoreMesh(
        core_axis_name="c", subcore_axis_name="s",
        num_cores=_NC, num_subcores=_NS)
    return pl.kernel(
        _sc_body,
        out_type=jax.ShapeDtypeStruct((_ROWS, _COLS), jnp.float32),
        mesh=mesh,
        scratch_types=[
            pltpu.VMEM((_XROWS, _XCOLS), jnp.float32),
            pltpu.VMEM((_KS * _KS, _L), jnp.float32),
            pltpu.VMEM((_RPW, _COLS), jnp.float32),
        ],
        compiler_params=pltpu.CompilerParams(use_tc_tiling_on_sc=False),
    )(xp, wvec)


def kernel(input, kernel):
    wvec = jnp.tile(kernel.reshape(_KS * _KS, 1), (1, _L))
    return _splat_sc(input, wvec)


# speedup vs baseline: 2.0009x; 2.0009x over previous
import jax
import jax.numpy as jnp
from jax import lax
from jax.experimental import pallas as pl
from jax.experimental.pallas import tpu as pltpu
from jax.experimental.pallas import tpu_sc as plsc

_ROWS = 512
_COLS = 512
_KS = 5
_WI = _KS // 2

_NC = 2
_NS = 16
_NW = _NC * _NS
_RPW = _ROWS // _NW
_L = 16
_NCHUNK = _COLS // _L
_XROWS = _RPW + 2 * _WI
_CPAD = 8
_XCOLS = _COLS + 2 * _CPAD


def _sc_body(x_hbm, w_hbm, o_hbm, xbuf, wbuf, obuf):
    cid = lax.axis_index("c")
    sid = lax.axis_index("s")
    wid = sid * _NC + cid
    row0 = wid * _RPW

    zero = jnp.zeros((_L,), jnp.float32)
    for t in range(_XROWS):
        xbuf[t, pl.ds(0, _L)] = zero
        xbuf[t, pl.ds(_XCOLS - _L, _L)] = zero

    pltpu.sync_copy(
        x_hbm.at[pl.ds(row0, _XROWS), :],
        xbuf.at[pl.ds(0, _XROWS), pl.ds(_CPAD, _COLS)])
    pltpu.sync_copy(w_hbm, wbuf)

    wv = [wbuf[i, :] for i in range(_KS * _KS)]

    @plsc.parallel_loop(0, _RPW * _NCHUNK, step=1, unroll=4)
    def _(i):
        r = i // _NCHUNK
        c = i % _NCHUNK
        terms = []
        for a in range(_KS):
            t = r + 2 * _WI - a
            for b in range(_KS):
                off = c * _L + _CPAD + _WI - b
                chunk = xbuf[t, pl.ds(off, _L)]
                terms.append(wv[a * _KS + b] * chunk)
        while len(terms) > 1:
            nxt = [terms[j] + terms[j + 1] for j in range(0, len(terms) - 1, 2)]
            if len(terms) % 2:
                nxt.append(terms[-1])
            terms = nxt
        obuf[r, pl.ds(c * _L, _L)] = terms[0]

    pltpu.sync_copy(obuf, o_hbm.at[pl.ds(row0, _RPW), :])


@jax.jit
def _splat_sc(x, wvec):
    xp = jnp.zeros((_ROWS + 2 * _WI, _COLS), jnp.float32)
    xp = lax.dynamic_update_slice(xp, x, (_WI, 0))
    mesh = plsc.VectorSubcoreMesh(
        core_axis_name="c", subcore_axis_name="s",
        num_cores=_NC, num_subcores=_NS)
    return pl.kernel(
        _sc_body,
        out_type=jax.ShapeDtypeStruct((_ROWS, _COLS), jnp.float32),
        mesh=mesh,
        scratch_types=[
            pltpu.VMEM((_XROWS, _XCOLS), jnp.float32),
            pltpu.VMEM((_KS * _KS, _L), jnp.float32),
            pltpu.VMEM((_RPW, _COLS), jnp.float32),
        ],
        compiler_params=pltpu.CompilerParams(use_tc_tiling_on_sc=False),
    )(xp, wvec)


def kernel(input, kernel):
    wvec = jnp.tile(kernel.reshape(_KS * _KS, 1), (1, _L))
    return _splat_sc(input, wvec)
